# baseline (speedup 1.0000x reference)
import functools

import jax
import jax.numpy as jnp
from jax import lax
from jax.experimental import pallas as pl
from jax.experimental.pallas import tpu as pltpu
from jax.experimental.pallas import tpu_sc as plsc

P = 16
NQ = 100

_B, _C, _H, _W = 16, 3, 512, 512
_CHUNKS = _B * _C * _H * (_W // 16)
_UNIT = 2048
_NW = 32
_UNITS_PER_W = _CHUNKS // _UNIT // _NW


def _patchify_sc(pix_ref, out_ref, sums_ref, buf, asm, stage, sem_l, sem_s):
    wid = lax.axis_index("c") * 16 + lax.axis_index("s")

    def issue_loads(t, slot):
        u = wid * 16 + t
        b = u // 32
        a = u % 32
        for c in range(3):
            row0 = (b * 3 + c) * 512 + a * 16
            pltpu.make_async_copy(
                pix_ref.at[pl.ds(row0, 16), :],
                buf.at[slot, pl.ds(c * 16, 16), :], sem_l).start()

    issue_loads(0, 0)
    zero = jnp.zeros((16,), jnp.float32)

    def body(t, acc):
        slot = lax.rem(t, 2)
        u = wid * 16 + t
        b = u // 32
        a = u % 32
        for c in range(3):
            pltpu.make_async_copy(
                pix_ref.at[pl.ds(0, 16), :],
                buf.at[slot, pl.ds(c * 16, 16), :], sem_l).wait()

        @pl.when(t + 1 < 16)
        def _():
            issue_loads(t + 1, lax.rem(t + 1, 2))

        @pl.when(t >= 2)
        def _():
            pltpu.make_async_copy(
                asm.at[slot], out_ref.at[pl.ds(0, 32), :], sem_s).wait()

        acc = list(acc)
        for bp in range(32):
            for c in range(3):
                for i in range(16):
                    v = buf[slot, c * 16 + i, pl.ds(bp * 16, 16)]
                    asm[slot, bp, pl.ds((c * 16 + i) * 16, 16)] = v
                    j = c * 2 + (i % 2)
                    acc[j] = acc[j] + v
        p0 = b * 1024 + a * 32
        pltpu.make_async_copy(
            asm.at[slot], out_ref.at[pl.ds(p0, 32), :], sem_s).start()
        return tuple(acc)

    acc = lax.fori_loop(0, 16, body, (zero,) * 6)
    for t in (14, 15):
        pltpu.make_async_copy(
            asm.at[t % 2], out_ref.at[pl.ds(0, 32), :], sem_s).wait()
    for c in range(3):
        stage[pl.ds(c * 16, 16)] = acc[2 * c] + acc[2 * c + 1]
    for c in range(3, 8):
        stage[pl.ds(c * 16, 16)] = zero
    pltpu.sync_copy(stage, sums_ref.at[wid, :])


def _router_kernel(sums_ref, rW_ref, rb_ref, probs_ref, choice_ref, loss_ref):
    s = sums_ref[:, :]
    pooled = s.reshape(16, 2, 8, 16).sum(axis=(1, 3))[:, :3] / (512.0 * 512.0)
    rW = rW_ref[:, :]
    logits = jnp.sum(pooled[:, None, :] * rW[None, :, :], axis=2) \
        + rb_ref[0, :][None, :]
    probs = jax.nn.softmax(logits, axis=1)
    probs_ref[:, :] = probs
    choice_ref[0, :] = jnp.argmax(logits, axis=1).astype(jnp.int32)
    e = rW.shape[0]
    usage = jnp.mean(probs, axis=0)
    loss_ref[:, :] = jnp.mean((usage - 1.0 / e) ** 2).reshape(1, 1)


def _expert_kernel(choices_ref, p_ref, w_ref, b_ref, wc_ref, wb_ref,
                   hid_ref, log_ref, box_ref):
    x = p_ref[0].astype(jnp.bfloat16)
    w = w_ref[0]
    h = jnp.dot(x, w, preferred_element_type=jnp.float32)
    h = h + b_ref[0, 0][None, :]
    h = jax.nn.gelu(h)
    hid_ref[0] = h
    q = h[:NQ, :]
    log_ref[0] = jnp.dot(q, wc_ref[0], preferred_element_type=jnp.float32)
    box_ref[0] = jax.nn.sigmoid(
        jnp.dot(q, wb_ref[0], preferred_element_type=jnp.float32))


def kernel(pixel_values, router_W, router_b, expert_patch_W, expert_patch_b,
           expert_cls_W, expert_box_W):
    B, C, H, W = pixel_values.shape
    E, D_in, D = expert_patch_W.shape
    NC = expert_cls_W.shape[2]
    nh, nw = H // P, W // P
    NP = nh * nw

    pix2d = pixel_values.reshape(B * C * H, W)
    patchify = functools.partial(
        pl.kernel,
        mesh=plsc.VectorSubcoreMesh(core_axis_name="c", subcore_axis_name="s"),
        out_type=[
            jax.ShapeDtypeStruct((B * NP, C * P * P), jnp.float32),
            jax.ShapeDtypeStruct((32, 128), jnp.float32),
        ],
        scratch_types=[
            pltpu.VMEM((2, 48, 512), jnp.float32),
            pltpu.VMEM((2, 32, 768), jnp.float32),
            pltpu.VMEM((128,), jnp.float32),
            pltpu.SemaphoreType.DMA,
            pltpu.SemaphoreType.DMA,
        ],
    )(_patchify_sc)
    patches_flat, sc_sums = patchify(pix2d)
    patches = patches_flat.reshape(B, NP, C * P * P)

    probs, choices2d, loss2d = pl.pallas_call(
        _router_kernel,
        grid=(1,),
        in_specs=[
            pl.BlockSpec((32, 128), lambda i: (0, 0)),
            pl.BlockSpec((E, C), lambda i: (0, 0)),
            pl.BlockSpec((1, E), lambda i: (0, 0)),
        ],
        out_specs=[
            pl.BlockSpec((B, E), lambda i: (0, 0)),
            pl.BlockSpec((1, B), lambda i: (0, 0)),
            pl.BlockSpec((1, 1), lambda i: (0, 0)),
        ],
        out_shape=[
            jax.ShapeDtypeStruct((B, E), jnp.float32),
            jax.ShapeDtypeStruct((1, B), jnp.int32),
            jax.ShapeDtypeStruct((1, 1), jnp.float32),
        ],
    )(sc_sums, router_W, router_b.reshape(1, E))
    choices = choices2d[0]
    routing_loss = loss2d[0, 0]

    patch_W16 = expert_patch_W.astype(jnp.bfloat16)

    bp3 = expert_patch_b.reshape(E, 1, D)
    grid_spec = pltpu.PrefetchScalarGridSpec(
        num_scalar_prefetch=1,
        grid=(B,),
        in_specs=[
            pl.BlockSpec((1, NP, D_in), lambda b, ch: (b, 0, 0)),
            pl.BlockSpec((1, D_in, D), lambda b, ch: (ch[b], 0, 0)),
            pl.BlockSpec((1, 1, D), lambda b, ch: (ch[b], 0, 0)),
            pl.BlockSpec((1, D, NC), lambda b, ch: (ch[b], 0, 0)),
            pl.BlockSpec((1, D, 4), lambda b, ch: (ch[b], 0, 0)),
        ],
        out_specs=[
            pl.BlockSpec((1, NP, D), lambda b, ch: (b, 0, 0)),
            pl.BlockSpec((1, NQ, NC), lambda b, ch: (b, 0, 0)),
            pl.BlockSpec((1, NQ, 4), lambda b, ch: (b, 0, 0)),
        ],
    )
    hidden, batch_logits, batch_pred_boxes = pl.pallas_call(
        _expert_kernel,
        grid_spec=grid_spec,
        out_shape=[
            jax.ShapeDtypeStruct((B, NP, D), jnp.float32),
            jax.ShapeDtypeStruct((B, NQ, NC), jnp.float32),
            jax.ShapeDtypeStruct((B, NQ, 4), jnp.float32),
        ],
    )(choices, patches, patch_W16, bp3, expert_cls_W, expert_box_W)

    return (batch_logits, batch_pred_boxes, hidden, probs, choices,
            routing_loss)
